# baseline (speedup 1.0000x reference)
import jax
import jax.numpy as jnp
from jax import lax
from jax.experimental import pallas as pl
from jax.experimental.pallas import tpu as pltpu
from jax.experimental.pallas import tpu_sc as plsc

_SUPPORT = 300
_EPS = 1e-3
_K = 601
_N = 131072
_NW = 32
_RPW = _N // _NW
_B = 64
_NBLK = _RPW // _B


def _sqrt1p(ax):
    bits = plsc.bitcast(ax, jnp.int32)
    y = plsc.bitcast(jnp.int32(0x5F3759DF) - (bits >> 1), jnp.float32)
    for _ in range(3):
        y = y * (1.5 - 0.5 * ax * y * y)
    return ax * y


def _encode16(xv):
    s = jnp.sign(xv) * (_sqrt1p(jnp.abs(xv) + 1.0) - 1.0) + _EPS * xv
    xc = jnp.minimum(jnp.maximum(s, -float(_SUPPORT)), float(_SUPPORT))
    u = xc + float(_SUPPORT)
    g = jnp.minimum(u.astype(jnp.int32), _K - 2)
    ud = u - g.astype(jnp.float32)
    ld = 1.0 - ud
    return g, ld, ud


def _sc_body(x_hbm, out_hbm, x_v, buf0, buf1, gsave, sem0, sem1):
    nc = 2
    wid = lax.axis_index("s") * nc + lax.axis_index("c")
    row0 = wid * _RPW
    bufs = (buf0, buf1)
    sems = (sem0, sem1)
    lanes = lax.iota(jnp.int32, 16)
    zeros = jnp.zeros((16,), jnp.float32)

    pltpu.sync_copy(x_hbm.at[pl.ds(row0, _RPW)], x_v)

    def _zero_row(r, _):
        for b in range(2):
            ref = bufs[b]
            for c in range(_K // 16):
                ref[r, pl.ds(c * 16, 16)] = zeros
            tail_idx = (_K // 16) * 16 + lanes
            tmask = tail_idx < _K
            plsc.store_scatter(
                ref, [jnp.full((16,), r, jnp.int32), tail_idx], zeros, mask=tmask
            )
        return 0

    lax.fori_loop(0, _B, _zero_row, 0)
    for s in range(2):
        for j in range(_B // 16):
            gsave[s, j] = jnp.zeros((16,), jnp.int32)

    def _fill(s, blk):
        ref = bufs[s]
        for j in range(_B // 16):
            xv = x_v[pl.ds(blk * _B + j * 16, 16)]
            g, ld, ud = _encode16(xv)
            rows = j * 16 + lanes
            plsc.store_scatter(ref, [rows, g], ld)
            plsc.store_scatter(ref, [rows, g + 1], ud)
            gsave[s, j] = g

    def _rezero(s):
        ref = bufs[s]
        for j in range(_B // 16):
            g = gsave[s, j]
            rows = j * 16 + lanes
            plsc.store_scatter(ref, [rows, g], zeros)
            plsc.store_scatter(ref, [rows, g + 1], zeros)

    def _dma_start(s, blk):
        base = row0 + blk * _B
        pltpu.async_copy(bufs[s], out_hbm.at[pl.ds(base, _B)], sems[s])

    def _dma_wait(s):
        pltpu.make_async_copy(bufs[s], out_hbm.at[pl.ds(row0, _B)], sems[s]).wait()

    _fill(0, 0)
    _dma_start(0, 0)
    _fill(1, 1)
    _dma_start(1, 1)

    def _pair(p, _):
        for s in range(2):
            blk = 2 * p + s
            _dma_wait(s)
            _rezero(s)
            _fill(s, blk)
            _dma_start(s, blk)
        return 0

    lax.fori_loop(1, _NBLK // 2, _pair, 0)

    _dma_wait(0)
    _dma_wait(1)


def kernel(x, atom_vector):
    mesh = plsc.VectorSubcoreMesh(core_axis_name="c", subcore_axis_name="s")
    f = pl.kernel(
        _sc_body,
        mesh=mesh,
        out_type=jax.ShapeDtypeStruct((_N, _K), jnp.float32),
        scratch_types=[
            pltpu.VMEM((_RPW,), jnp.float32),
            pltpu.VMEM((_B, _K), jnp.float32),
            pltpu.VMEM((_B, _K), jnp.float32),
            pltpu.VMEM((2, _B // 16, 16), jnp.int32),
            pltpu.SemaphoreType.DMA,
            pltpu.SemaphoreType.DMA,
        ],
        compiler_params=pltpu.CompilerParams(
            needs_layout_passes=False, use_tc_tiling_on_sc=True
        ),
    )
    return f(x)


# speedup vs baseline: 3.3196x; 3.3196x over previous
import jax
import jax.numpy as jnp
from jax import lax
from jax.experimental import pallas as pl
from jax.experimental.pallas import tpu as pltpu
from jax.experimental.pallas import tpu_sc as plsc

_SUPPORT = 300
_EPS = 1e-3
_K = 601
_N = 131072
_NW = 32
_CPW = _N // _NW
_C = 128
_NBLK = _CPW // _C


def _sqrt1p(ax):
    bits = plsc.bitcast(ax, jnp.int32)
    y = plsc.bitcast(jnp.int32(0x5F3759DF) - (bits >> 1), jnp.float32)
    for _ in range(3):
        y = y * (1.5 - 0.5 * ax * y * y)
    return ax * y


def _encode16(xv):
    s = jnp.sign(xv) * (_sqrt1p(jnp.abs(xv) + 1.0) - 1.0) + _EPS * xv
    xc = jnp.minimum(jnp.maximum(s, -float(_SUPPORT)), float(_SUPPORT))
    u = xc + float(_SUPPORT)
    g = jnp.minimum(u.astype(jnp.int32), _K - 2)
    ud = u - g.astype(jnp.float32)
    ld = 1.0 - ud
    return g, ld, ud


def _sc_body(x_hbm, out_hbm, x_v, buf, gsave):
    nc = 2
    wid = lax.axis_index("s") * nc + lax.axis_index("c")
    col0 = wid * _CPW
    lanes = lax.iota(jnp.int32, 16)
    zeros = jnp.zeros((16,), jnp.float32)

    pltpu.sync_copy(x_hbm.at[pl.ds(col0, _CPW)], x_v)

    def _zero_row(r, _):
        for c in range(_C // 16):
            buf[r, pl.ds(c * 16, 16)] = zeros
        return 0

    lax.fori_loop(0, _K, _zero_row, 0)

    def _block(blk, _):
        for j in range(_C // 16):
            xv = x_v[pl.ds(blk * _C + j * 16, 16)]
            g, ld, ud = _encode16(xv)
            cols = j * 16 + lanes
            plsc.store_scatter(buf, [g, cols], ld)
            plsc.store_scatter(buf, [g + 1, cols], ud)
            gsave[j] = g
        pltpu.sync_copy(buf, out_hbm.at[:, pl.ds(col0 + blk * _C, _C)])
        for j in range(_C // 16):
            g = gsave[j]
            cols = j * 16 + lanes
            plsc.store_scatter(buf, [g, cols], zeros)
            plsc.store_scatter(buf, [g + 1, cols], zeros)
        return 0

    lax.fori_loop(0, _NBLK, _block, 0)


def kernel(x, atom_vector):
    mesh = plsc.VectorSubcoreMesh(core_axis_name="c", subcore_axis_name="s")
    f = pl.kernel(
        _sc_body,
        mesh=mesh,
        out_type=jax.ShapeDtypeStruct((_K, _N), jnp.float32),
        scratch_types=[
            pltpu.VMEM((_CPW,), jnp.float32),
            pltpu.VMEM((_K, _C), jnp.float32),
            pltpu.VMEM((_C // 16, 16), jnp.int32),
        ],
        compiler_params=pltpu.CompilerParams(
            needs_layout_passes=False, use_tc_tiling_on_sc=True
        ),
    )
    return f(x).T
